# Initial kernel scaffold; baseline (speedup 1.0000x reference)
import jax
import jax.numpy as jnp
from jax.experimental import pallas as pl


def kernel(x, weight):
    raise NotImplementedError("write your pallas kernel here")


# speedup vs baseline: 107.4775x; 107.4775x over previous
import functools

import jax
import jax.numpy as jnp
from jax import lax
from jax.experimental import pallas as pl
from jax.experimental.pallas import tpu as pltpu
from jax.experimental.pallas import tpu_sc as plsc

NC = 2
NS = 16
LANES = 16
NW = NC * NS


@functools.partial(jax.jit, static_argnums=(2, 3, 4, 5))
def _embedding_bag_mean(x_flat, wt_flat, B, L, E, D):
    chunk = B // NW
    groups = chunk // LANES

    mesh = plsc.VectorSubcoreMesh(core_axis_name="c", subcore_axis_name="s")

    @functools.partial(
        pl.kernel,
        out_type=jax.ShapeDtypeStruct((B * D,), jnp.float32),
        mesh=mesh,
        scratch_types=[
            pltpu.VMEM((chunk * L,), jnp.int32),
            pltpu.VMEM((chunk * D,), jnp.float32),
            pltpu.VMEM((D * E,), jnp.float32),
        ],
        compiler_params=pltpu.CompilerParams(needs_layout_passes=False),
    )
    def sc_kernel(x_hbm, wt_hbm, out_hbm, x_v, out_v, wt_v):
        wid = lax.axis_index("s") * NC + lax.axis_index("c")
        base = wid * chunk
        pltpu.sync_copy(wt_hbm, wt_v)
        pltpu.sync_copy(x_hbm.at[pl.ds(base * L, chunk * L)], x_v)

        lane = lax.iota(jnp.int32, LANES)
        dim_off = [jnp.full((LANES,), d * E, jnp.int32) for d in range(D)]
        scale = jnp.float32(1.0 / L)

        def group_body(g, _):
            rows = g * LANES + lane
            flat_base = rows * L

            def pos_body(l, accs):
                xv = plsc.load_gather(x_v, [flat_base + l])
                return tuple(
                    accs[d] + plsc.load_gather(wt_v, [dim_off[d] + xv])
                    for d in range(D)
                )

            accs = lax.fori_loop(
                0, L, pos_body,
                tuple(jnp.zeros((LANES,), jnp.float32) for _ in range(D)),
            )
            out_base = rows * D
            for d in range(D):
                plsc.store_scatter(out_v, [out_base + d], accs[d] * scale)
            return 0

        lax.fori_loop(0, groups, group_body, 0)
        pltpu.sync_copy(out_v, out_hbm.at[pl.ds(base * D, chunk * D)])

    return sc_kernel(x_flat, wt_flat)


def kernel(x, weight):
    B, L = x.shape
    E, D = weight.shape
    x_flat = x.astype(jnp.int32).reshape(-1)
    wt_flat = weight.T.astype(jnp.float32).reshape(-1)
    out = _embedding_bag_mean(x_flat, wt_flat, B, L, E, D)
    return out.reshape(B, D)


# speedup vs baseline: 125.7695x; 1.1702x over previous
import functools

import jax
import jax.numpy as jnp
from jax import lax
from jax.experimental import pallas as pl
from jax.experimental.pallas import tpu as pltpu
from jax.experimental.pallas import tpu_sc as plsc

NC = 2
NS = 16
LANES = 16
NW = NC * NS
UNROLL = 4


def _table_layout(E):
    soff = ((E * E + 7) // 8) * 8
    stride = ((soff + E + 7) // 8) * 8
    return soff, stride


@functools.partial(jax.jit, static_argnums=(2, 3, 4, 5))
def _embedding_bag_mean(x_flat, pt_flat, B, L, E, D):
    chunk = B // NW
    groups = chunk // LANES
    npairs = L // 2
    tail = L - 2 * npairs
    P2 = E * E
    SOFF, STRIDE = _table_layout(E)

    mesh = plsc.VectorSubcoreMesh(core_axis_name="c", subcore_axis_name="s")

    @functools.partial(
        pl.kernel,
        out_type=jax.ShapeDtypeStruct((B * D,), jnp.float32),
        mesh=mesh,
        scratch_types=[
            pltpu.VMEM((chunk * L,), jnp.int32),
            pltpu.VMEM((chunk * D,), jnp.float32),
            pltpu.VMEM((D * STRIDE,), jnp.float32),
        ],
        compiler_params=pltpu.CompilerParams(needs_layout_passes=False),
    )
    def sc_kernel(x_hbm, pt_hbm, out_hbm, x_v, out_v, pt_v):
        wid = lax.axis_index("s") * NC + lax.axis_index("c")
        base = wid * chunk
        pltpu.sync_copy(pt_hbm, pt_v)
        pltpu.sync_copy(x_hbm.at[pl.ds(base * L, chunk * L)], x_v)

        psub = [pt_v.at[pl.ds(d * STRIDE, P2)] for d in range(D)]
        ssub = [pt_v.at[pl.ds(d * STRIDE + SOFF, E)] for d in range(D)]

        lane = lax.iota(jnp.int32, LANES)
        scale = jnp.float32(1.0 / L)
        e_vec = jnp.full((LANES,), E, jnp.int32)

        def group_body(g, _):
            rows = g * LANES + lane
            flat_base = rows * L

            def pair_body(p, accs):
                accs = list(accs)
                pos0 = flat_base + 2 * p * UNROLL
                for u in range(UNROLL):
                    xv1 = plsc.load_gather(x_v, [pos0 + (2 * u)])
                    xv2 = plsc.load_gather(x_v, [pos0 + (2 * u + 1)])
                    pidx = xv1 * e_vec + xv2
                    for d in range(D):
                        accs[d] = accs[d] + plsc.load_gather(psub[d], [pidx])
                return tuple(accs)

            accs = lax.fori_loop(
                0, npairs // UNROLL, pair_body,
                tuple(jnp.zeros((LANES,), jnp.float32) for _ in range(D)),
            )
            accs = list(accs)
            done = (npairs // UNROLL) * UNROLL
            for p in range(done, npairs):
                xv1 = plsc.load_gather(x_v, [flat_base + (2 * p)])
                xv2 = plsc.load_gather(x_v, [flat_base + (2 * p + 1)])
                pidx = xv1 * e_vec + xv2
                for d in range(D):
                    accs[d] = accs[d] + plsc.load_gather(psub[d], [pidx])
            if tail:
                xv = plsc.load_gather(x_v, [flat_base + (L - 1)])
                for d in range(D):
                    accs[d] = accs[d] + plsc.load_gather(ssub[d], [xv])

            out_base = rows * D
            for d in range(D):
                plsc.store_scatter(out_v, [out_base + d], accs[d] * scale)
            return 0

        lax.fori_loop(0, groups, group_body, 0)
        pltpu.sync_copy(out_v, out_hbm.at[pl.ds(base * D, chunk * D)])

    return sc_kernel(x_flat, pt_flat)


def kernel(x, weight):
    B, L = x.shape
    E, D = weight.shape
    x_flat = x.astype(jnp.int32).reshape(-1)
    w = weight.astype(jnp.float32)
    soff, stride = _table_layout(E)
    pairs = (w[:, None, :] + w[None, :, :]).reshape(E * E, D)
    pt = (
        jnp.zeros((D, stride), jnp.float32)
        .at[:, : E * E].set(pairs.T)
        .at[:, soff: soff + E].set(w.T)
        .reshape(-1)
    )
    out = _embedding_bag_mean(x_flat, pt, B, L, E, D)
    return out.reshape(B, D)
